# baseline (speedup 1.0000x reference)
import functools

import jax
import jax.numpy as jnp
from jax import lax
from jax.experimental import pallas as pl
from jax.experimental.pallas import tpu as pltpu
from jax.experimental.pallas import tpu_sc as plsc

_VOCAB = 100000
_D = 32
_B = 1024


def _sc_gather(emb_table, x):
    info = plsc.get_sparse_core_info()
    nw = info.num_cores * info.num_subcores
    b_per_w = _B // nw
    mesh = plsc.VectorSubcoreMesh(core_axis_name="c", subcore_axis_name="s")

    @functools.partial(
        pl.kernel,
        mesh=mesh,
        out_type=jax.ShapeDtypeStruct((_B, _D), jnp.float32),
        scratch_types=[
            pltpu.VMEM((b_per_w,), jnp.int32),
            pltpu.VMEM((b_per_w, _D), jnp.float32),
            pltpu.SemaphoreType.DMA,
        ],
        compiler_params=pltpu.CompilerParams(use_tc_tiling_on_sc=False),
    )
    def gather_kernel(table_hbm, idx_hbm, out_hbm, idx_v, rows_v, sem):
        wid = lax.axis_index("s") * info.num_cores + lax.axis_index("c")
        base = wid * b_per_w
        pltpu.sync_copy(idx_hbm.at[pl.ds(base, b_per_w)], idx_v)
        pltpu.async_copy(table_hbm.at[idx_v], rows_v, sem).wait()
        pltpu.sync_copy(rows_v, out_hbm.at[pl.ds(base, b_per_w)])

    return gather_kernel(emb_table, x)


_TN = 2048
_NBUF = 4
_NFULL = _VOCAB // _TN


def _dot(h, w, b):
    return (
        lax.dot_general(
            h, w, (((1,), (1,)), ((), ())), preferred_element_type=jnp.float32
        )
        + b
    )


def _out_copy(acc, out_hbm, sems, step, slot):
    return pltpu.make_async_copy(
        acc.at[slot],
        out_hbm.at[:, pl.ds(step * _TN, _TN)],
        sems.at[slot],
    )


def _interior_kernel(h_ref, w_ref, b_ref, out_hbm, acc, sems):
    i = pl.program_id(0)
    slot = lax.rem(i, _NBUF)

    @pl.when(i >= _NBUF)
    def _():
        _out_copy(acc, out_hbm, sems, i - _NBUF, slot).wait()

    acc[slot] = _dot(h_ref[...], w_ref[...], b_ref[...])
    _out_copy(acc, out_hbm, sems, i, slot).start()

    @pl.when(i == _NFULL - 1)
    def _():
        for k in range(_NBUF):
            j = _NFULL - _NBUF + k
            _out_copy(acc, out_hbm, sems, j, j % _NBUF).wait()


def _tail_kernel(h_ref, w_ref, b_ref, prev_ref, out_ref):
    del prev_ref
    out_ref[...] = _dot(h_ref[...], w_ref[...], b_ref[...])


def _projection(h, W, b):
    b2 = b.reshape(1, _VOCAB)
    interior = pl.pallas_call(
        _interior_kernel,
        grid=(_NFULL,),
        in_specs=[
            pl.BlockSpec((_B, _D), lambda i: (0, 0)),
            pl.BlockSpec((_TN, _D), lambda i: (i, 0)),
            pl.BlockSpec((1, _TN), lambda i: (0, i)),
        ],
        out_specs=pl.BlockSpec(memory_space=pl.ANY),
        out_shape=jax.ShapeDtypeStruct((_B, _VOCAB), jnp.float32),
        scratch_shapes=[
            pltpu.VMEM((_NBUF, _B, _TN), jnp.float32),
            pltpu.SemaphoreType.DMA((_NBUF,)),
        ],
    )(h, W, b2)
    return pl.pallas_call(
        _tail_kernel,
        grid=(1,),
        in_specs=[
            pl.BlockSpec((_B, _D), lambda i: (0, 0)),
            pl.BlockSpec((_TN, _D), lambda i: (_NFULL, 0)),
            pl.BlockSpec((1, _TN), lambda i: (0, _NFULL)),
            pl.BlockSpec(memory_space=pl.ANY),
        ],
        out_specs=pl.BlockSpec((_B, _TN), lambda i: (0, _NFULL)),
        out_shape=jax.ShapeDtypeStruct((_B, _VOCAB), jnp.float32),
        input_output_aliases={3: 0},
    )(h, W, b2, interior)


def kernel(x, emb_table, W, b):
    h = _sc_gather(emb_table, x)
    return _projection(h, W, b)


# speedup vs baseline: 1.0826x; 1.0826x over previous
import functools

import jax
import jax.numpy as jnp
from jax import lax
from jax.experimental import pallas as pl
from jax.experimental.pallas import tpu as pltpu
from jax.experimental.pallas import tpu_sc as plsc

_VOCAB = 100000
_D = 32
_B = 1024


def _sc_gather(emb_table, x):
    info = plsc.get_sparse_core_info()
    nw = info.num_cores * info.num_subcores
    b_per_w = _B // nw
    mesh = plsc.VectorSubcoreMesh(core_axis_name="c", subcore_axis_name="s")

    @functools.partial(
        pl.kernel,
        mesh=mesh,
        out_type=jax.ShapeDtypeStruct((_B, _D), jnp.float32),
        scratch_types=[
            pltpu.VMEM((b_per_w,), jnp.int32),
            pltpu.VMEM((b_per_w, _D), jnp.float32),
            pltpu.SemaphoreType.DMA,
        ],
        compiler_params=pltpu.CompilerParams(use_tc_tiling_on_sc=False),
    )
    def gather_kernel(table_hbm, idx_hbm, out_hbm, idx_v, rows_v, sem):
        wid = lax.axis_index("s") * info.num_cores + lax.axis_index("c")
        base = wid * b_per_w
        pltpu.sync_copy(idx_hbm.at[pl.ds(base, b_per_w)], idx_v)
        pltpu.async_copy(table_hbm.at[idx_v], rows_v, sem).wait()
        pltpu.sync_copy(rows_v, out_hbm.at[pl.ds(base, b_per_w)])

    return gather_kernel(emb_table, x)


_BM = 32


def _proj_kernel(h_ref, wt_ref, b_ref, out_ref):
    out_ref[...] = (
        lax.dot_general(
            h_ref[...],
            wt_ref[...],
            (((1,), (0,)), ((), ())),
            preferred_element_type=jnp.float32,
        )
        + b_ref[...]
    )


def _projection(h, W, b):
    return pl.pallas_call(
        _proj_kernel,
        grid=(_B // _BM,),
        in_specs=[
            pl.BlockSpec((_BM, _D), lambda i: (i, 0)),
            pl.BlockSpec((_D, _VOCAB), lambda i: (0, 0)),
            pl.BlockSpec((1, _VOCAB), lambda i: (0, 0)),
        ],
        out_specs=pl.BlockSpec((_BM, _VOCAB), lambda i: (i, 0)),
        out_shape=jax.ShapeDtypeStruct((_B, _VOCAB), jnp.float32),
    )(h, W.T, b.reshape(1, _VOCAB))


def kernel(x, emb_table, W, b):
    h = _sc_gather(emb_table, x)
    return _projection(h, W, b)


# speedup vs baseline: 1.0833x; 1.0006x over previous
import functools

import jax
import jax.numpy as jnp
from jax import lax
from jax.experimental import pallas as pl
from jax.experimental.pallas import tpu as pltpu
from jax.experimental.pallas import tpu_sc as plsc

_VOCAB = 100000
_D = 32
_B = 1024


def _sc_gather(emb_table, x):
    info = plsc.get_sparse_core_info()
    nw = info.num_cores * info.num_subcores
    b_per_w = _B // nw
    mesh = plsc.VectorSubcoreMesh(core_axis_name="c", subcore_axis_name="s")

    @functools.partial(
        pl.kernel,
        mesh=mesh,
        out_type=jax.ShapeDtypeStruct((_B, _D), jnp.float32),
        scratch_types=[
            pltpu.VMEM((b_per_w,), jnp.int32),
            pltpu.VMEM((b_per_w, _D), jnp.float32),
            pltpu.SemaphoreType.DMA,
        ],
        compiler_params=pltpu.CompilerParams(use_tc_tiling_on_sc=False),
    )
    def gather_kernel(table_hbm, idx_hbm, out_hbm, idx_v, rows_v, sem):
        wid = lax.axis_index("s") * info.num_cores + lax.axis_index("c")
        base = wid * b_per_w
        pltpu.sync_copy(idx_hbm.at[pl.ds(base, b_per_w)], idx_v)
        pltpu.async_copy(table_hbm.at[idx_v], rows_v, sem).wait()
        pltpu.sync_copy(rows_v, out_hbm.at[pl.ds(base, b_per_w)])

    return gather_kernel(emb_table, x)


_BM = 32
_NS = _B // _BM
_NBUF = 2
_NSPLIT = 4
_RS = _BM // _NSPLIT


def _slab_copy(acc, out_hbm, sems, step, slot, r):
    return pltpu.make_async_copy(
        acc.at[slot, pl.ds(r * _RS, _RS), :],
        out_hbm.at[pl.ds(step * _BM + r * _RS, _RS), :],
        sems.at[slot, r],
    )


def _proj_kernel(h_ref, wt_ref, b_ref, out_hbm, acc, sems):
    i = pl.program_id(0)
    slot = lax.rem(i, _NBUF)

    @pl.when(i >= _NBUF)
    def _():
        for r in range(_NSPLIT):
            _slab_copy(acc, out_hbm, sems, i - _NBUF, slot, r).wait()

    acc[slot] = (
        lax.dot_general(
            h_ref[...],
            wt_ref[...],
            (((1,), (0,)), ((), ())),
            preferred_element_type=jnp.float32,
        )
        + b_ref[...]
    )
    for r in range(_NSPLIT):
        _slab_copy(acc, out_hbm, sems, i, slot, r).start()

    @pl.when(i == _NS - 1)
    def _():
        for k in range(_NBUF):
            j = _NS - _NBUF + k
            for r in range(_NSPLIT):
                _slab_copy(acc, out_hbm, sems, j, j % _NBUF, r).wait()


def _projection(h, W, b):
    return pl.pallas_call(
        _proj_kernel,
        grid=(_NS,),
        in_specs=[
            pl.BlockSpec((_BM, _D), lambda i: (i, 0)),
            pl.BlockSpec((_D, _VOCAB), lambda i: (0, 0)),
            pl.BlockSpec((1, _VOCAB), lambda i: (0, 0)),
        ],
        out_specs=pl.BlockSpec(memory_space=pl.ANY),
        out_shape=jax.ShapeDtypeStruct((_B, _VOCAB), jnp.float32),
        scratch_shapes=[
            pltpu.VMEM((_NBUF, _BM, _VOCAB), jnp.float32),
            pltpu.SemaphoreType.DMA((_NBUF, _NSPLIT)),
        ],
    )(h, W.T, b.reshape(1, _VOCAB))


def kernel(x, emb_table, W, b):
    h = _sc_gather(emb_table, x)
    return _projection(h, W, b)
